# Initial kernel scaffold; baseline (speedup 1.0000x reference)
import functools

import jax
import jax.numpy as jnp
from jax import lax
from jax.experimental import pallas as pl
from jax.experimental.pallas import tpu as pltpu
from jax.experimental.pallas import tpu_sc as plsc

NC = 2
NS = 16
NW = NC * NS

CHUNK = 1600


def _gather_body(nchunk, table_hbm, idx_hbm, out_hbm, idx_v, rows_v, sem):
    c = lax.axis_index("c")
    s = lax.axis_index("s")
    wid = s * NC + c
    base = wid * (nchunk * CHUNK)

    @pl.loop(0, nchunk)
    def _(g):
        off = pl.multiple_of(base + g * CHUNK, 8)
        pltpu.sync_copy(idx_hbm.at[pl.ds(off, CHUNK)], idx_v)
        pltpu.async_copy(table_hbm.at[idx_v], rows_v, sem).wait()
        pltpu.sync_copy(rows_v, out_hbm.at[pl.ds(off, CHUNK)])


@functools.partial(jax.jit, static_argnames=("n", "d"))
def _sc_gather(table, flat_idx, n, d):
    assert n % (NW * CHUNK) == 0
    nchunk = n // (NW * CHUNK)
    mesh = plsc.VectorSubcoreMesh(
        core_axis_name="c", subcore_axis_name="s", num_cores=NC, num_subcores=NS
    )
    return pl.kernel(
        functools.partial(_gather_body, nchunk),
        out_type=jax.ShapeDtypeStruct((n, d), table.dtype),
        mesh=mesh,
        scratch_types=[
            pltpu.VMEM((CHUNK,), jnp.int32),
            pltpu.VMEM((CHUNK, d), table.dtype),
            pltpu.SemaphoreType.DMA,
        ],
    )(table, flat_idx)


def kernel(indices, table):
    b, h = indices.shape
    d = table.shape[1]
    flat = indices.reshape(b * h).astype(jnp.int32)
    out = _sc_gather(table, flat, b * h, d)
    return out.reshape(b, h, d)


# speedup vs baseline: 1.1027x; 1.1027x over previous
import functools

import jax
import jax.numpy as jnp
from jax import lax
from jax.experimental import pallas as pl
from jax.experimental.pallas import tpu as pltpu
from jax.experimental.pallas import tpu_sc as plsc

NC = 2
NS = 16
NW = NC * NS

CHUNK = 1600


def _gather_body(nchunk, table_hbm, idx_hbm, out_hbm, idx_v, rows_v, sem):
    c = lax.axis_index("c")
    s = lax.axis_index("s")
    wid = s * NC + c
    base = wid * (nchunk * CHUNK)

    @pl.loop(0, nchunk)
    def _(g):
        off = pl.multiple_of(base + g * CHUNK, 8)
        pltpu.sync_copy(idx_hbm.at[pl.ds(off, CHUNK)], idx_v)
        pltpu.async_copy(table_hbm.at[idx_v], rows_v, sem).wait()
        pltpu.sync_copy(rows_v, out_hbm.at[pl.ds(off, CHUNK)])


@functools.partial(jax.jit, static_argnames=("n", "d"))
def _sc_gather(table, flat_idx, n, d):
    assert n % (NW * CHUNK) == 0
    nchunk = n // (NW * CHUNK)
    mesh = plsc.VectorSubcoreMesh(
        core_axis_name="c", subcore_axis_name="s", num_cores=NC, num_subcores=NS
    )
    return pl.kernel(
        functools.partial(_gather_body, nchunk),
        out_type=jax.ShapeDtypeStruct((n, d), table.dtype),
        mesh=mesh,
        scratch_types=[
            pltpu.VMEM((CHUNK,), jnp.int32),
            pltpu.VMEM((CHUNK, d), table.dtype),
            pltpu.SemaphoreType.DMA,
        ],
        compiler_params=pltpu.CompilerParams(use_tc_tiling_on_sc=False),
    )(table, flat_idx)


def kernel(indices, table):
    b, h = indices.shape
    d = table.shape[1]
    flat = indices.reshape(b * h).astype(jnp.int32)
    out = _sc_gather(table, flat, b * h, d)
    return out.reshape(b, h, d)


# speedup vs baseline: 1.1095x; 1.0062x over previous
import functools

import jax
import jax.numpy as jnp
from jax import lax
from jax.experimental import pallas as pl
from jax.experimental.pallas import tpu as pltpu
from jax.experimental.pallas import tpu_sc as plsc

NC = 2
NS = 16
NW = NC * NS

CHUNK = 1600


def _gather_body(nchunk, table_hbm, idx_hbm, out_hbm,
                 idx_v, rows0, rows1, semg0, semg1, semw0, semw1):
    c = lax.axis_index("c")
    s = lax.axis_index("s")
    wid = s * NC + c
    rows_per_w = nchunk * CHUNK
    base = pl.multiple_of(wid * rows_per_w, 8)

    pltpu.sync_copy(idx_hbm.at[pl.ds(base, rows_per_w)], idx_v)

    bufs = (rows0, rows1)
    semg = (semg0, semg1)
    semw = (semw0, semw1)

    def idx_slice(g):
        return idx_v.at[pl.ds(pl.multiple_of(g * CHUNK, 8), CHUNK)]

    def start_gather(g, slot):
        pltpu.async_copy(table_hbm.at[idx_slice(g)], bufs[slot], semg[slot])

    def wait_gather(g, slot):
        pltpu.make_async_copy(table_hbm.at[idx_slice(g)], bufs[slot],
                              semg[slot]).wait()

    def out_slice(g):
        return out_hbm.at[pl.ds(pl.multiple_of(base + g * CHUNK, 8), CHUNK)]

    def start_write(g, slot):
        pltpu.async_copy(bufs[slot], out_slice(g), semw[slot])

    def wait_write(g, slot):
        pltpu.make_async_copy(bufs[slot], out_slice(g), semw[slot]).wait()

    start_gather(0, 0)
    start_gather(1, 1)

    @pl.loop(0, nchunk - 2, step=2)
    def _(g):
        wait_gather(g, 0)
        start_write(g, 0)
        wait_gather(g + 1, 1)
        start_write(g + 1, 1)
        wait_write(g, 0)
        start_gather(g + 2, 0)
        wait_write(g + 1, 1)
        start_gather(g + 3, 1)

    g = nchunk - 2
    wait_gather(g, 0)
    start_write(g, 0)
    wait_gather(g + 1, 1)
    start_write(g + 1, 1)
    wait_write(g, 0)
    wait_write(g + 1, 1)


@functools.partial(jax.jit, static_argnames=("n", "d"))
def _sc_gather(table, flat_idx, n, d):
    assert n % (NW * CHUNK) == 0 and (n // (NW * CHUNK)) % 2 == 0
    nchunk = n // (NW * CHUNK)
    mesh = plsc.VectorSubcoreMesh(
        core_axis_name="c", subcore_axis_name="s", num_cores=NC, num_subcores=NS
    )
    return pl.kernel(
        functools.partial(_gather_body, nchunk),
        out_type=jax.ShapeDtypeStruct((n, d), table.dtype),
        mesh=mesh,
        scratch_types=[
            pltpu.VMEM((nchunk * CHUNK,), jnp.int32),
            pltpu.VMEM((CHUNK, d), table.dtype),
            pltpu.VMEM((CHUNK, d), table.dtype),
            pltpu.SemaphoreType.DMA,
            pltpu.SemaphoreType.DMA,
            pltpu.SemaphoreType.DMA,
            pltpu.SemaphoreType.DMA,
        ],
        compiler_params=pltpu.CompilerParams(use_tc_tiling_on_sc=False),
    )(table, flat_idx)


def kernel(indices, table):
    b, h = indices.shape
    d = table.shape[1]
    flat = indices.reshape(b * h).astype(jnp.int32)
    out = _sc_gather(table, flat, b * h, d)
    return out.reshape(b, h, d)


# speedup vs baseline: 1.6442x; 1.4820x over previous
import functools

import jax
import jax.numpy as jnp
from jax import lax
from jax.experimental import pallas as pl
from jax.experimental.pallas import tpu as pltpu
from jax.experimental.pallas import tpu_sc as plsc

NC = 2
NS = 16
NW = NC * NS

BBLK = 512
TPC = BBLK // 128


def _emb_body(hist, d, table_hbm, idx_hbm, out_hbm,
              idx_v, g0, g1, r0, r1, semg0, semg1, semw0, semw1):
    c = lax.axis_index("c")
    s = lax.axis_index("s")
    wid = s * NC + c
    ct = d // 8

    pltpu.sync_copy(idx_hbm.at[:, pl.ds(wid * BBLK, BBLK)], idx_v)

    bufs = (g0, g1)
    rbufs = (r0, r1)
    semg = (semg0, semg1)
    semw = (semw0, semw1)

    def start_gather(h, slot):
        pltpu.async_copy(table_hbm.at[idx_v.at[h]], bufs[slot], semg[slot])

    def wait_gather(h, slot):
        pltpu.make_async_copy(table_hbm.at[idx_v.at[h]], bufs[slot],
                              semg[slot]).wait()

    def start_write(h, slot):
        for i in range(ct):
            pltpu.async_copy(rbufs[slot].at[i],
                             out_hbm.at[h * ct + i, pl.ds(wid * TPC, TPC)],
                             semw[slot])

    def wait_write(h, slot):
        for i in range(ct):
            pltpu.make_async_copy(rbufs[slot].at[i],
                                  out_hbm.at[h * ct + i, pl.ds(wid * TPC, TPC)],
                                  semw[slot]).wait()

    iota = lax.iota(jnp.int32, 16)
    cols = [jnp.full((16,), cc, jnp.int32) for cc in range(d)]

    def transpose_chunk(slot):
        g = bufs[slot]
        r = rbufs[slot]

        @pl.loop(0, TPC)
        def _(tl):
            base = tl * 128
            for blk in range(8):
                rows = base + blk * 16 + iota
                for i in range(ct):
                    for cl in range(8):
                        r[i, tl, cl, pl.ds(blk * 16, 16)] = plsc.load_gather(
                            g, [rows, cols[i * 8 + cl]])

    start_gather(0, 0)
    start_gather(1, 1)

    @pl.loop(0, hist // 2)
    def _(p):
        h = p * 2
        for slot in (0, 1):
            hh = h + slot
            wait_gather(hh, slot)

            @pl.when(p > 0)
            def _():
                wait_write(hh - 2, slot)

            transpose_chunk(slot)
            start_write(hh, slot)

            @pl.when(hh + 2 < hist)
            def _():
                start_gather(hh + 2, slot)

    wait_write(hist - 2, 0)
    wait_write(hist - 1, 1)


@functools.partial(jax.jit, static_argnames=("b", "hist", "d"))
def _sc_emb(table, idx_t, b, hist, d):
    assert b % (NW * BBLK) == 0 and hist % 2 == 0 and d % 8 == 0
    ct = d // 8
    mesh = plsc.VectorSubcoreMesh(
        core_axis_name="c", subcore_axis_name="s", num_cores=NC, num_subcores=NS
    )
    ltile = pl.kernel(
        functools.partial(_emb_body, hist, d),
        out_type=jax.ShapeDtypeStruct((hist * ct, b // 128, 8, 128),
                                      table.dtype),
        mesh=mesh,
        scratch_types=[
            pltpu.VMEM((hist, BBLK), jnp.int32),
            pltpu.VMEM((BBLK, d), table.dtype),
            pltpu.VMEM((BBLK, d), table.dtype),
            pltpu.VMEM((ct, TPC, 8, 128), table.dtype),
            pltpu.VMEM((ct, TPC, 8, 128), table.dtype),
            pltpu.SemaphoreType.DMA,
            pltpu.SemaphoreType.DMA,
            pltpu.SemaphoreType.DMA,
            pltpu.SemaphoreType.DMA,
        ],
        compiler_params=pltpu.CompilerParams(
            use_tc_tiling_on_sc=False, needs_layout_passes=False),
    )(table, idx_t)
    l5 = ltile.reshape(hist, ct, b // 128, 8, 128)
    return jnp.transpose(l5, (2, 4, 0, 1, 3)).reshape(b, hist, d)


def kernel(indices, table):
    b, h = indices.shape
    d = table.shape[1]
    idx_t = jnp.transpose(indices).astype(jnp.int32)
    return _sc_emb(table, idx_t, b, h, d)


# speedup vs baseline: 2.1117x; 1.2843x over previous
import functools

import jax
import jax.numpy as jnp
from jax import lax
from jax.experimental import pallas as pl
from jax.experimental.pallas import tpu as pltpu
from jax.experimental.pallas import tpu_sc as plsc

NC = 2
NS = 16
NW = NC * NS

BBLK = 512
TPC = BBLK // 128


def _emb_body(hist, d, table_hbm, idx_hbm, out_hbm,
              idx_v, g0, g1, r0, r1, semg0, semg1, semw0, semw1):
    c = lax.axis_index("c")
    s = lax.axis_index("s")
    wid = s * NC + c
    ct = d // 8

    pltpu.sync_copy(idx_hbm.at[:, pl.ds(wid * BBLK, BBLK)], idx_v)

    bufs = (g0, g1)
    rbufs = (r0, r1)
    semg = (semg0, semg1)
    semw = (semw0, semw1)

    def start_gather(h, slot):
        pltpu.async_copy(table_hbm.at[idx_v.at[h]],
                         bufs[slot], semg[slot])

    def wait_gather(h, slot):
        pltpu.make_async_copy(table_hbm.at[idx_v.at[h]],
                              bufs[slot], semg[slot]).wait()

    def start_write(h, slot):
        for i in range(ct):
            pltpu.async_copy(rbufs[slot].at[i],
                             out_hbm.at[h * ct + i, pl.ds(wid * TPC, TPC)],
                             semw[slot])

    def wait_write(h, slot):
        for i in range(ct):
            pltpu.make_async_copy(rbufs[slot].at[i],
                                  out_hbm.at[h * ct + i, pl.ds(wid * TPC, TPC)],
                                  semw[slot]).wait()

    iota = lax.iota(jnp.int32, 16)
    cols = [jnp.full((16,), cc, jnp.int32) for cc in range(d)]

    def transpose_chunk(slot):
        g = bufs[slot]
        r = rbufs[slot]

        @plsc.parallel_loop(0, TPC * 8, unroll=2)
        def _(tb):
            tl = tb // 8
            blk = tb % 8
            rows = tl * 128 + blk * 16 + iota
            for i in range(ct):
                for cl in range(8):
                    r[i, tl, cl, pl.ds(blk * 16, 16)] = plsc.load_gather(
                        g, [rows, cols[i * 8 + cl]])

    start_gather(0, 0)
    start_gather(1, 1)

    @pl.loop(0, hist // 2)
    def _(p):
        h = p * 2
        for slot in (0, 1):
            hh = h + slot
            wait_gather(hh, slot)

            @pl.when(p > 0)
            def _():
                wait_write(hh - 2, slot)

            transpose_chunk(slot)
            start_write(hh, slot)

            @pl.when(hh + 2 < hist)
            def _():
                start_gather(hh + 2, slot)

    wait_write(hist - 2, 0)
    wait_write(hist - 1, 1)


@functools.partial(jax.jit, static_argnames=("b", "hist", "d"))
def _sc_emb(table, idx_t, b, hist, d):
    assert b % (NW * BBLK) == 0 and hist % 2 == 0 and d % 8 == 0
    ct = d // 8
    mesh = plsc.VectorSubcoreMesh(
        core_axis_name="c", subcore_axis_name="s", num_cores=NC, num_subcores=NS
    )
    ltile = pl.kernel(
        functools.partial(_emb_body, hist, d),
        out_type=jax.ShapeDtypeStruct((hist * ct, b // 128, 8, 128),
                                      table.dtype),
        mesh=mesh,
        scratch_types=[
            pltpu.VMEM((hist, BBLK), jnp.int32),
            pltpu.VMEM((BBLK, d), table.dtype),
            pltpu.VMEM((BBLK, d), table.dtype),
            pltpu.VMEM((ct, TPC, 8, 128), table.dtype),
            pltpu.VMEM((ct, TPC, 8, 128), table.dtype),
            pltpu.SemaphoreType.DMA,
            pltpu.SemaphoreType.DMA,
            pltpu.SemaphoreType.DMA,
            pltpu.SemaphoreType.DMA,
        ],
        compiler_params=pltpu.CompilerParams(
            use_tc_tiling_on_sc=False, needs_layout_passes=False),
    )(table, idx_t)
    l5 = ltile.reshape(hist, ct, b // 128, 8, 128)
    return jnp.transpose(l5, (2, 4, 0, 1, 3)).reshape(b, hist, d)


def kernel(indices, table):
    b, h = indices.shape
    d = table.shape[1]
    idx_t = jnp.transpose(indices).astype(jnp.int32)
    return _sc_emb(table, idx_t, b, h, d)
